# baseline (speedup 1.0000x reference)
import jax
import jax.numpy as jnp
from jax import lax
from jax.experimental import pallas as pl
from jax.experimental.pallas import tpu as pltpu
from jax.experimental.pallas import tpu_sc as plsc

N = 49688
D = 128
B = 8
P = 512
E = B * P
NW = 32
EPW = E // NW
RBLK = 8192
NB = -(-N // RBLK)
NPAD = NB * RBLK
TAIL = N - (NB - 1) * RBLK
NBUF = 6
L = 16

_SC_MESH = plsc.VectorSubcoreMesh(
    core_axis_name="c", subcore_axis_name="s", num_cores=2, num_subcores=16)
_SC_PARAMS = pltpu.CompilerParams(needs_layout_passes=False)


def _dedup_body(nodes_ref, xsrc_ref):
    q_iota = lax.broadcasted_iota(jnp.int32, (P, P), 1)
    for bb in range(B):
        row = nodes_ref[bb, :]
        eq = row[:, None] == row[None, :]
        lastq = jnp.max(jnp.where(eq, q_iota, -1), axis=1)
        xsrc_ref[bb, :] = lastq + bb * P


def _dedup(nodes):
    return pl.pallas_call(
        _dedup_body,
        out_shape=jax.ShapeDtypeStruct((B, P), jnp.int32),
    )(nodes)


def _sc_compute_body(nodes_hbm, xsrc_hbm, x_hbm, table_hbm, alpha_hbm,
                     upd_hbm, fidx_hbm,
                     idx_v, xsrc_v, fidx_v, aidx_v, tbl_v, x_v, ar_v, a_v,
                     sem1, sem2, sem3):
    c = lax.axis_index("c")
    s = lax.axis_index("s")
    wid = s * 2 + c
    base = wid * EPW
    pltpu.sync_copy(nodes_hbm.at[pl.ds(base, EPW)], idx_v)
    pltpu.sync_copy(xsrc_hbm.at[pl.ds(base, EPW)], xsrc_v)
    boff = (base // P) * N
    for j in range(EPW // L):
        sl = pl.ds(j * L, L)
        v = idx_v[sl]
        fidx_v[sl] = v + boff
        aidx_v[sl] = lax.shift_right_logical(v, 7)
    cp1 = pltpu.async_copy(table_hbm.at[idx_v], tbl_v, sem1)
    cp2 = pltpu.async_copy(x_hbm.at[xsrc_v], x_v, sem2)
    cp3 = pltpu.async_copy(alpha_hbm.at[aidx_v], ar_v, sem3)
    pltpu.sync_copy(fidx_v, fidx_hbm.at[pl.ds(base, EPW)])
    cp3.wait()
    for g in range(EPW // L):
        sl = pl.ds(g * L, L)
        ent = lax.iota(jnp.int32, L) + g * L
        cols = idx_v[sl] & 127
        a_v[sl] = plsc.load_gather(ar_v, [ent, cols])
    cp1.wait()
    cp2.wait()

    def row_body(i, carry):
        a = plsc.load_gather(a_v, [jnp.full((L,), i, jnp.int32)])
        for j in range(D // L):
            sl = pl.ds(j * L, L)
            t = tbl_v[i, sl]
            xx = x_v[i, sl]
            tbl_v[i, sl] = t + a * (xx - t)
        return carry

    lax.fori_loop(0, EPW, row_body, 0)
    pltpu.sync_copy(tbl_v, upd_hbm.at[pl.ds(base, EPW)])


_sc_compute = pl.kernel(
    _sc_compute_body,
    out_type=(
        jax.ShapeDtypeStruct((E, D), jnp.float32),
        jax.ShapeDtypeStruct((E,), jnp.int32),
    ),
    mesh=_SC_MESH,
    compiler_params=_SC_PARAMS,
    scratch_types=[
        pltpu.VMEM((EPW,), jnp.int32),
        pltpu.VMEM((EPW,), jnp.int32),
        pltpu.VMEM((EPW,), jnp.int32),
        pltpu.VMEM((EPW,), jnp.int32),
        pltpu.VMEM((EPW, D), jnp.float32),
        pltpu.VMEM((EPW, D), jnp.float32),
        pltpu.VMEM((EPW, D), jnp.float32),
        pltpu.VMEM((EPW,), jnp.float32),
        pltpu.SemaphoreType.DMA,
        pltpu.SemaphoreType.DMA,
        pltpu.SemaphoreType.DMA,
    ],
)


def _blk(j):
    return TAIL if j == NB - 1 else RBLK


def _dense_body(tbl_hbm, out_hbm, buf, insem, outsem):
    def start_in(j):
        k = j % NBUF
        pltpu.make_async_copy(
            tbl_hbm.at[pl.ds(j * RBLK, _blk(j))],
            buf.at[k, pl.ds(0, _blk(j))], insem.at[k]).start()

    def wait_in(j):
        k = j % NBUF
        pltpu.make_async_copy(
            tbl_hbm.at[pl.ds(j * RBLK, _blk(j))],
            buf.at[k, pl.ds(0, _blk(j))], insem.at[k]).wait()

    def out_cps(j):
        k = j % NBUF
        return [
            pltpu.make_async_copy(
                buf.at[k, pl.ds(0, _blk(j))],
                out_hbm.at[b, pl.ds(j * RBLK, _blk(j))], outsem.at[k])
            for b in range(B)
        ]

    for j in range(min(NBUF - 2, NB)):
        start_in(j)
    for j in range(NB):
        wait_in(j)
        for cp in out_cps(j):
            cp.start()
        if j - 2 >= 0 and j + NBUF - 2 < NB:
            for cp in out_cps(j - 2):
                cp.wait()
        if j + NBUF - 2 < NB:
            start_in(j + NBUF - 2)
    for j in range(max(NB - NBUF, 0), NB):
        for cp in out_cps(j):
            cp.wait()


def _dense_copy(table):
    return pl.pallas_call(
        _dense_body,
        in_specs=[pl.BlockSpec(memory_space=pl.ANY)],
        out_specs=pl.BlockSpec(memory_space=pl.ANY),
        out_shape=jax.ShapeDtypeStruct((B, N, D), jnp.float32),
        scratch_shapes=[
            pltpu.VMEM((NBUF, RBLK, D), jnp.float32),
            pltpu.SemaphoreType.DMA((NBUF,)),
            pltpu.SemaphoreType.DMA((NBUF,)),
        ],
    )(table)


def _sc_scatter_body(out_hbm, upd_hbm, fidx_hbm, fidx_v, upd_v, sem1):
    c = lax.axis_index("c")
    s = lax.axis_index("s")
    wid = s * 2 + c
    base = wid * EPW
    pltpu.sync_copy(fidx_hbm.at[pl.ds(base, EPW)], fidx_v)
    cp = pltpu.async_copy(upd_hbm.at[pl.ds(base, EPW)], upd_v, sem1)
    cp.wait()
    pltpu.async_copy(upd_v, out_hbm.at[fidx_v], sem1).wait()


_sc_scatter = pl.kernel(
    _sc_scatter_body,
    out_type=(),
    mesh=_SC_MESH,
    compiler_params=_SC_PARAMS,
    scratch_types=[
        pltpu.VMEM((EPW,), jnp.int32),
        pltpu.VMEM((EPW, D), jnp.float32),
        pltpu.SemaphoreType.DMA,
    ],
)


def kernel(nodes, x, table, alpha):
    xsrc = _dedup(nodes)
    alpha2d = jnp.pad(alpha.reshape(N), (0, NPAD - N)).reshape(NPAD // D, D)
    upd, fidx = _sc_compute(nodes.reshape(E), xsrc.reshape(E), x, table,
                            alpha2d)
    dense = _dense_copy(table)
    out_ref = jax.new_ref(dense.reshape(B * N, D))
    _sc_scatter(out_ref, upd, fidx)
    return out_ref[...].reshape(B, N, D)


# speedup vs baseline: 1.0860x; 1.0860x over previous
import jax
import jax.numpy as jnp
from jax import lax
from jax.experimental import pallas as pl
from jax.experimental.pallas import tpu as pltpu
from jax.experimental.pallas import tpu_sc as plsc

N = 49688
D = 128
B = 8
P = 512
E = B * P
NW = 32
EPW = E // NW
RBLK = 8192
NB = -(-N // RBLK)
NPAD = NB * RBLK
TAIL = N - (NB - 1) * RBLK
NBUF = 6
L = 16

_SC_MESH = plsc.VectorSubcoreMesh(
    core_axis_name="c", subcore_axis_name="s", num_cores=2, num_subcores=16)
_SC_PARAMS = pltpu.CompilerParams(needs_layout_passes=False)


def _sc_compute_body(nodes_hbm, x_hbm, table_hbm, alpha_hbm,
                     upd_hbm, fidx_hbm,
                     nsamp_v, idx_v, xsrc_v, fidx_v, aidx_v, tbl_v, x_v, ar_v,
                     a_v, sem1, sem2, sem3):
    c = lax.axis_index("c")
    s = lax.axis_index("s")
    wid = s * 2 + c
    base = wid * EPW
    sb = base // P
    off = base - sb * P
    pltpu.sync_copy(nodes_hbm.at[sb], nsamp_v)
    boff = sb * N
    for j in range(EPW // L):
        sl = pl.ds(j * L, L)
        v = nsamp_v[pl.ds(off + j * L, L)]
        idx_v[sl] = v
        fidx_v[sl] = v + boff
        aidx_v[sl] = lax.shift_right_logical(v, 7)
    pltpu.sync_copy(fidx_v, fidx_hbm.at[pl.ds(base, EPW)])

    ent = [idx_v[pl.ds(g * L, L)] for g in range(EPW // L)]

    def q_body(q, lq):
        nq = plsc.load_gather(nsamp_v, [jnp.full((L,), q, jnp.int32)])
        return tuple(
            jnp.where(ent[g] == nq, q, lq[g]) for g in range(EPW // L))

    lq = lax.fori_loop(0, P, q_body,
                       tuple(jnp.zeros((L,), jnp.int32)
                             for _ in range(EPW // L)))
    xoff = sb * P
    for g in range(EPW // L):
        xsrc_v[pl.ds(g * L, L)] = lq[g] + xoff

    cp1 = pltpu.async_copy(table_hbm.at[idx_v], tbl_v, sem1)
    cp2 = pltpu.async_copy(x_hbm.at[xsrc_v], x_v, sem2)
    cp3 = pltpu.async_copy(alpha_hbm.at[aidx_v], ar_v, sem3)
    cp3.wait()
    for g in range(EPW // L):
        sl = pl.ds(g * L, L)
        ent = lax.iota(jnp.int32, L) + g * L
        cols = idx_v[sl] & 127
        a_v[sl] = plsc.load_gather(ar_v, [ent, cols])
    cp1.wait()
    cp2.wait()

    def row_body(i, carry):
        a = plsc.load_gather(a_v, [jnp.full((L,), i, jnp.int32)])
        for j in range(D // L):
            sl = pl.ds(j * L, L)
            t = tbl_v[i, sl]
            xx = x_v[i, sl]
            tbl_v[i, sl] = t + a * (xx - t)
        return carry

    lax.fori_loop(0, EPW, row_body, 0)
    pltpu.sync_copy(tbl_v, upd_hbm.at[pl.ds(base, EPW)])


_sc_compute = pl.kernel(
    _sc_compute_body,
    out_type=(
        jax.ShapeDtypeStruct((E, D), jnp.float32),
        jax.ShapeDtypeStruct((E,), jnp.int32),
    ),
    mesh=_SC_MESH,
    compiler_params=_SC_PARAMS,
    scratch_types=[
        pltpu.VMEM((P,), jnp.int32),
        pltpu.VMEM((EPW,), jnp.int32),
        pltpu.VMEM((EPW,), jnp.int32),
        pltpu.VMEM((EPW,), jnp.int32),
        pltpu.VMEM((EPW,), jnp.int32),
        pltpu.VMEM((EPW, D), jnp.float32),
        pltpu.VMEM((EPW, D), jnp.float32),
        pltpu.VMEM((EPW, D), jnp.float32),
        pltpu.VMEM((EPW,), jnp.float32),
        pltpu.SemaphoreType.DMA,
        pltpu.SemaphoreType.DMA,
        pltpu.SemaphoreType.DMA,
    ],
)


def _blk(j):
    return TAIL if j == NB - 1 else RBLK


def _dense_body(tbl_hbm, out_hbm, buf, insem, outsem):
    def start_in(j):
        k = j % NBUF
        pltpu.make_async_copy(
            tbl_hbm.at[pl.ds(j * RBLK, _blk(j))],
            buf.at[k, pl.ds(0, _blk(j))], insem.at[k]).start()

    def wait_in(j):
        k = j % NBUF
        pltpu.make_async_copy(
            tbl_hbm.at[pl.ds(j * RBLK, _blk(j))],
            buf.at[k, pl.ds(0, _blk(j))], insem.at[k]).wait()

    def out_cps(j):
        k = j % NBUF
        return [
            pltpu.make_async_copy(
                buf.at[k, pl.ds(0, _blk(j))],
                out_hbm.at[b, pl.ds(j * RBLK, _blk(j))], outsem.at[k])
            for b in range(B)
        ]

    for j in range(min(NBUF - 2, NB)):
        start_in(j)
    for j in range(NB):
        wait_in(j)
        for cp in out_cps(j):
            cp.start()
        if j - 2 >= 0 and j + NBUF - 2 < NB:
            for cp in out_cps(j - 2):
                cp.wait()
        if j + NBUF - 2 < NB:
            start_in(j + NBUF - 2)
    for j in range(max(NB - NBUF, 0), NB):
        for cp in out_cps(j):
            cp.wait()


def _dense_copy(table):
    return pl.pallas_call(
        _dense_body,
        in_specs=[pl.BlockSpec(memory_space=pl.ANY)],
        out_specs=pl.BlockSpec(memory_space=pl.ANY),
        out_shape=jax.ShapeDtypeStruct((B, N, D), jnp.float32),
        scratch_shapes=[
            pltpu.VMEM((NBUF, RBLK, D), jnp.float32),
            pltpu.SemaphoreType.DMA((NBUF,)),
            pltpu.SemaphoreType.DMA((NBUF,)),
        ],
    )(table)


def _sc_scatter_body(out_hbm, upd_hbm, fidx_hbm, fidx_v, upd_v, sem1):
    c = lax.axis_index("c")
    s = lax.axis_index("s")
    wid = s * 2 + c
    base = wid * EPW
    pltpu.sync_copy(fidx_hbm.at[pl.ds(base, EPW)], fidx_v)
    cp = pltpu.async_copy(upd_hbm.at[pl.ds(base, EPW)], upd_v, sem1)
    cp.wait()
    pltpu.async_copy(upd_v, out_hbm.at[fidx_v], sem1).wait()


_sc_scatter = pl.kernel(
    _sc_scatter_body,
    out_type=(),
    mesh=_SC_MESH,
    compiler_params=_SC_PARAMS,
    scratch_types=[
        pltpu.VMEM((EPW,), jnp.int32),
        pltpu.VMEM((EPW, D), jnp.float32),
        pltpu.SemaphoreType.DMA,
    ],
)


def kernel(nodes, x, table, alpha):
    alpha2d = jnp.pad(alpha.reshape(N), (0, NPAD - N)).reshape(NPAD // D, D)
    upd, fidx = _sc_compute(nodes, x, table, alpha2d)
    dense = _dense_copy(table)
    out_ref = jax.new_ref(dense.reshape(B * N, D))
    _sc_scatter(out_ref, upd, fidx)
    return out_ref[...].reshape(B, N, D)


# speedup vs baseline: 1.0916x; 1.0051x over previous
import jax
import jax.numpy as jnp
from jax import lax
from jax.experimental import pallas as pl
from jax.experimental.pallas import tpu as pltpu
from jax.experimental.pallas import tpu_sc as plsc

N = 49688
D = 128
B = 8
P = 512
E = B * P
NW = 32
EPW = E // NW
RBLK = 8192
NB = -(-N // RBLK)
NPAD = NB * RBLK
TAIL = N - (NB - 1) * RBLK
NBUF = 6
L = 16

_SC_MESH = plsc.VectorSubcoreMesh(
    core_axis_name="c", subcore_axis_name="s", num_cores=2, num_subcores=16)
_SC_PARAMS = pltpu.CompilerParams(needs_layout_passes=False)


def _sc_compute_body(nodes_hbm, x_hbm, table_hbm, alpha_hbm,
                     upd_hbm, fidx_hbm,
                     nsamp_v, idx_v, xsrc_v, fidx_v, aidx_v, tbl_v, x_v, ar_v,
                     a_v, sem1, sem2, sem3):
    c = lax.axis_index("c")
    s = lax.axis_index("s")
    wid = s * 2 + c
    base = wid * EPW
    sb = base // P
    off = base - sb * P
    pltpu.sync_copy(nodes_hbm.at[sb], nsamp_v)
    boff = sb * N
    for j in range(EPW // L):
        sl = pl.ds(j * L, L)
        v = nsamp_v[pl.ds(off + j * L, L)]
        idx_v[sl] = v
        fidx_v[sl] = v + boff
        aidx_v[sl] = lax.shift_right_logical(v, 7)
    pltpu.sync_copy(fidx_v, fidx_hbm.at[pl.ds(base, EPW)])

    ent = [idx_v[pl.ds(g * L, L)] for g in range(EPW // L)]

    def q_body(q, lq):
        nq = plsc.load_gather(nsamp_v, [jnp.full((L,), q, jnp.int32)])
        return tuple(
            jnp.where(ent[g] == nq, q, lq[g]) for g in range(EPW // L))

    lq = lax.fori_loop(0, P, q_body,
                       tuple(jnp.zeros((L,), jnp.int32)
                             for _ in range(EPW // L)))
    xoff = sb * P
    for g in range(EPW // L):
        xsrc_v[pl.ds(g * L, L)] = lq[g] + xoff

    cp1 = pltpu.async_copy(table_hbm.at[idx_v], tbl_v, sem1)
    cp2 = pltpu.async_copy(x_hbm.at[xsrc_v], x_v, sem2)
    cp3 = pltpu.async_copy(alpha_hbm.at[aidx_v], ar_v, sem3)
    cp3.wait()
    for g in range(EPW // L):
        sl = pl.ds(g * L, L)
        ent = lax.iota(jnp.int32, L) + g * L
        cols = idx_v[sl] & 127
        a_v[sl] = plsc.load_gather(ar_v, [ent, cols])
    cp1.wait()
    cp2.wait()

    def row_body(i, carry):
        a = plsc.load_gather(a_v, [jnp.full((L,), i, jnp.int32)])
        for j in range(D // L):
            sl = pl.ds(j * L, L)
            t = tbl_v[i, sl]
            xx = x_v[i, sl]
            tbl_v[i, sl] = t + a * (xx - t)
        return carry

    lax.fori_loop(0, EPW, row_body, 0)
    pltpu.sync_copy(tbl_v, upd_hbm.at[pl.ds(base, EPW)])


_sc_compute = pl.kernel(
    _sc_compute_body,
    out_type=(
        jax.ShapeDtypeStruct((E, D), jnp.float32),
        jax.ShapeDtypeStruct((E,), jnp.int32),
    ),
    mesh=_SC_MESH,
    compiler_params=_SC_PARAMS,
    scratch_types=[
        pltpu.VMEM((P,), jnp.int32),
        pltpu.VMEM((EPW,), jnp.int32),
        pltpu.VMEM((EPW,), jnp.int32),
        pltpu.VMEM((EPW,), jnp.int32),
        pltpu.VMEM((EPW,), jnp.int32),
        pltpu.VMEM((EPW, D), jnp.float32),
        pltpu.VMEM((EPW, D), jnp.float32),
        pltpu.VMEM((EPW, D), jnp.float32),
        pltpu.VMEM((EPW,), jnp.float32),
        pltpu.SemaphoreType.DMA,
        pltpu.SemaphoreType.DMA,
        pltpu.SemaphoreType.DMA,
    ],
)


def _blk(j):
    return TAIL if j == NB - 1 else RBLK


def _dense_body(tbl_hbm, out_hbm, buf, insem, outsem):
    def start_in(j):
        k = j % NBUF
        pltpu.make_async_copy(
            tbl_hbm.at[pl.ds(j * RBLK, _blk(j))],
            buf.at[k, pl.ds(0, _blk(j))], insem.at[k]).start()

    def wait_in(j):
        k = j % NBUF
        pltpu.make_async_copy(
            tbl_hbm.at[pl.ds(j * RBLK, _blk(j))],
            buf.at[k, pl.ds(0, _blk(j))], insem.at[k]).wait()

    def out_cps(j):
        k = j % NBUF
        return [
            pltpu.make_async_copy(
                buf.at[k, pl.ds(0, _blk(j))],
                out_hbm.at[b, pl.ds(j * RBLK, _blk(j))], outsem.at[k])
            for b in range(B)
        ]

    for j in range(min(NBUF - 2, NB)):
        start_in(j)
    for j in range(NB):
        wait_in(j)
        for cp in out_cps(j):
            cp.start()
        if j - 2 >= 0 and j + NBUF - 2 < NB:
            for cp in out_cps(j - 2):
                cp.wait()
        if j + NBUF - 2 < NB:
            start_in(j + NBUF - 2)
    for j in range(max(NB - NBUF, 0), NB):
        for cp in out_cps(j):
            cp.wait()


def _dense_copy(table):
    return pl.pallas_call(
        _dense_body,
        in_specs=[pl.BlockSpec(memory_space=pl.ANY)],
        out_specs=pl.BlockSpec(memory_space=pl.ANY),
        out_shape=jax.ShapeDtypeStruct((B, N, D), jnp.float32),
        scratch_shapes=[
            pltpu.VMEM((NBUF, RBLK, D), jnp.float32),
            pltpu.SemaphoreType.DMA((NBUF,)),
            pltpu.SemaphoreType.DMA((NBUF,)),
        ],
    )(table)


def _sc_scatter_body(out_hbm, upd_hbm, fidx_hbm, fidx_v, upd_v, sem1, sem2):
    c = lax.axis_index("c")
    s = lax.axis_index("s")
    wid = s * 2 + c
    base = wid * EPW
    cp = pltpu.async_copy(upd_hbm.at[pl.ds(base, EPW)], upd_v, sem1)
    pltpu.sync_copy(fidx_hbm.at[pl.ds(base, EPW)], fidx_v)
    cp.wait()
    pltpu.async_copy(upd_v, out_hbm.at[fidx_v], sem2).wait()


_sc_scatter = pl.kernel(
    _sc_scatter_body,
    out_type=(),
    mesh=_SC_MESH,
    compiler_params=_SC_PARAMS,
    scratch_types=[
        pltpu.VMEM((EPW,), jnp.int32),
        pltpu.VMEM((EPW, D), jnp.float32),
        pltpu.SemaphoreType.DMA,
        pltpu.SemaphoreType.DMA,
    ],
)


def kernel(nodes, x, table, alpha):
    alpha2d = jnp.pad(alpha.reshape(N), (0, NPAD - N)).reshape(NPAD // D, D)
    upd, fidx = _sc_compute(nodes, x, table, alpha2d)
    dense = _dense_copy(table)
    out_ref = jax.new_ref(dense.reshape(B * N, D))
    _sc_scatter(out_ref, upd, fidx)
    return out_ref[...].reshape(B, N, D)
